# baseline (speedup 1.0000x reference)
import functools

import jax
import jax.numpy as jnp
from jax import lax
from jax.experimental import pallas as pl
from jax.experimental.pallas import tpu as pltpu
from jax.experimental.pallas import tpu_sc as plsc

_VOCAB = 100000
_DIM = 1024
_BATCH = 4
_SEQ = 4096
_NTOK = _BATCH * _SEQ

_NC = 2
_NS = 16
_NW = _NC * _NS
_TOK_PER_W = _NTOK // _NW
_CHUNK = 8
_NCHUNK = _TOK_PER_W // _CHUNK
_NBUF = 8
_NGROUP = _NCHUNK // _NBUF


_IDX_HEAD = 2 * _NBUF * _CHUNK


def _emb_body(idx_hbm, table_hbm, out_hbm, idx_v, rows_v, isem,
              gsem0, gsem1, gsem2, gsem3, gsem4, gsem5, gsem6, gsem7,
              ssem0, ssem1, ssem2, ssem3, ssem4, ssem5, ssem6, ssem7):
    wid = lax.axis_index("s") * _NC + lax.axis_index("c")
    wper = _SEQ // _TOK_PER_W
    row = wid // wper
    col = (wid % wper) * _TOK_PER_W
    pltpu.sync_copy(idx_hbm.at[row, pl.ds(col, _IDX_HEAD)],
                    idx_v.at[pl.ds(0, _IDX_HEAD)])
    idx_rest = pltpu.async_copy(
        idx_hbm.at[row, pl.ds(col + _IDX_HEAD, _TOK_PER_W - _IDX_HEAD)],
        idx_v.at[pl.ds(_IDX_HEAD, _TOK_PER_W - _IDX_HEAD)], isem)

    gsems = (gsem0, gsem1, gsem2, gsem3, gsem4, gsem5, gsem6, gsem7)
    ssems = (ssem0, ssem1, ssem2, ssem3, ssem4, ssem5, ssem6, ssem7)

    def gdesc(j, b):
        return pltpu.make_async_copy(
            table_hbm.at[idx_v.at[pl.ds(j * _CHUNK, _CHUNK)]],
            rows_v.at[b], gsems[b])

    def sdesc(j, b):
        return pltpu.make_async_copy(
            rows_v.at[b],
            out_hbm.at[row, pl.ds(col + j * _CHUNK, _CHUNK)],
            ssems[b])

    for b in range(_NBUF - 1):
        gdesc(b, b).start()

    for b in range(_NBUF):
        gdesc(b, b).wait()
        pb = (b - 1) % _NBUF
        if b > 0:
            sdesc(b - 1, pb).wait()
        gdesc(b + _NBUF - 1, pb).start()
        sdesc(b, b).start()

    idx_rest.wait()

    @pl.loop(1, _NGROUP - 1)
    def _steady(g):
        j0 = g * _NBUF
        for b in range(_NBUF):
            j = j0 + b
            pb = (b - 1) % _NBUF
            gdesc(j, b).wait()
            sdesc(j - 1, pb).wait()
            gdesc(j + _NBUF - 1, pb).start()
            sdesc(j, b).start()

    j0 = (_NGROUP - 1) * _NBUF
    for b in range(_NBUF):
        j = j0 + b
        pb = (b - 1) % _NBUF
        gdesc(j, b).wait()
        sdesc(j - 1, pb).wait()
        if b == 0:
            gdesc(j + _NBUF - 1, pb).start()
        sdesc(j, b).start()
    sdesc(_NCHUNK - 1, (_NBUF - 1) % _NBUF).wait()


@functools.partial(jax.jit, static_argnames=())
def _emb(idx2d, weight):
    mesh = plsc.VectorSubcoreMesh(core_axis_name="c", subcore_axis_name="s")
    kern = pl.kernel(
        _emb_body,
        out_type=jax.ShapeDtypeStruct((_BATCH, _SEQ, _DIM), jnp.float32),
        mesh=mesh,
        scratch_types=[
            pltpu.VMEM((_TOK_PER_W,), jnp.int32),
            pltpu.VMEM((_NBUF, _CHUNK, _DIM), jnp.float32),
            pltpu.SemaphoreType.DMA,
            pltpu.SemaphoreType.DMA,
            pltpu.SemaphoreType.DMA,
            pltpu.SemaphoreType.DMA,
            pltpu.SemaphoreType.DMA,
            pltpu.SemaphoreType.DMA,
            pltpu.SemaphoreType.DMA,
            pltpu.SemaphoreType.DMA,
            pltpu.SemaphoreType.DMA,
            pltpu.SemaphoreType.DMA,
            pltpu.SemaphoreType.DMA,
            pltpu.SemaphoreType.DMA,
            pltpu.SemaphoreType.DMA,
            pltpu.SemaphoreType.DMA,
            pltpu.SemaphoreType.DMA,
            pltpu.SemaphoreType.DMA,
            pltpu.SemaphoreType.DMA,
        ],
    )
    return kern(idx2d, weight)


def kernel(indices, weight):
    return _emb(indices.astype(jnp.int32), weight)


# speedup vs baseline: 1.0042x; 1.0042x over previous
import functools

import jax
import jax.numpy as jnp
from jax import lax
from jax.experimental import pallas as pl
from jax.experimental.pallas import tpu as pltpu
from jax.experimental.pallas import tpu_sc as plsc

_VOCAB = 100000
_DIM = 1024
_BATCH = 4
_SEQ = 4096
_NTOK = _BATCH * _SEQ

_NC = 2
_NS = 16
_NW = _NC * _NS
_TOK_PER_W = _NTOK // _NW
_CHUNK = 16
_NCHUNK = _TOK_PER_W // _CHUNK
_NBUF = 4
_NGROUP = _NCHUNK // _NBUF
_IDX_HEAD = 2 * _NBUF * _CHUNK


def _emb_body(idx_hbm, table_hbm, out_hbm, idx_v, rows_v, isem,
              gsem0, gsem1, gsem2, gsem3, ssem0, ssem1, ssem2, ssem3):
    wid = lax.axis_index("s") * _NC + lax.axis_index("c")
    wper = _SEQ // _TOK_PER_W
    row = wid // wper
    col = (wid % wper) * _TOK_PER_W
    pltpu.sync_copy(idx_hbm.at[row, pl.ds(col, _IDX_HEAD)],
                    idx_v.at[pl.ds(0, _IDX_HEAD)])
    idx_rest = pltpu.async_copy(
        idx_hbm.at[row, pl.ds(col + _IDX_HEAD, _TOK_PER_W - _IDX_HEAD)],
        idx_v.at[pl.ds(_IDX_HEAD, _TOK_PER_W - _IDX_HEAD)], isem)

    gsems = (gsem0, gsem1, gsem2, gsem3)
    ssems = (ssem0, ssem1, ssem2, ssem3)

    def gdesc(j, b):
        return pltpu.make_async_copy(
            table_hbm.at[idx_v.at[pl.ds(j * _CHUNK, _CHUNK)]],
            rows_v.at[b], gsems[b])

    def sdesc(j, b):
        return pltpu.make_async_copy(
            rows_v.at[b],
            out_hbm.at[row, pl.ds(col + j * _CHUNK, _CHUNK)],
            ssems[b])

    for b in range(_NBUF - 1):
        gdesc(b, b).start()
    idx_rest.wait()

    @pl.loop(0, _NGROUP)
    def _lap(g):
        j0 = g * _NBUF
        for b in range(_NBUF):
            j = j0 + b
            pb = (b - 1) % _NBUF
            gdesc(j, b).wait()
            if b == 0:
                @pl.when(g > 0)
                def _():
                    sdesc(j - 1, pb).wait()

                gdesc(j + _NBUF - 1, pb).start()
            else:
                sdesc(j - 1, pb).wait()

                @pl.when(g < _NGROUP - 1)
                def _():
                    gdesc(j + _NBUF - 1, pb).start()
            sdesc(j, b).start()

    sdesc(_NCHUNK - 1, (_NBUF - 1) % _NBUF).wait()


@functools.partial(jax.jit, static_argnames=())
def _emb(idx2d, weight):
    mesh = plsc.VectorSubcoreMesh(core_axis_name="c", subcore_axis_name="s")
    kern = pl.kernel(
        _emb_body,
        out_type=jax.ShapeDtypeStruct((_BATCH, _SEQ, _DIM), jnp.float32),
        mesh=mesh,
        scratch_types=[
            pltpu.VMEM((_TOK_PER_W,), jnp.int32),
            pltpu.VMEM((_NBUF, _CHUNK, _DIM), jnp.float32),
            pltpu.SemaphoreType.DMA,
            pltpu.SemaphoreType.DMA,
            pltpu.SemaphoreType.DMA,
            pltpu.SemaphoreType.DMA,
            pltpu.SemaphoreType.DMA,
            pltpu.SemaphoreType.DMA,
            pltpu.SemaphoreType.DMA,
            pltpu.SemaphoreType.DMA,
            pltpu.SemaphoreType.DMA,
        ],
    )
    return kern(idx2d, weight)


def kernel(indices, weight):
    return _emb(indices.astype(jnp.int32), weight)


# speedup vs baseline: 1.0086x; 1.0044x over previous
import functools

import jax
import jax.numpy as jnp
from jax import lax
from jax.experimental import pallas as pl
from jax.experimental.pallas import tpu as pltpu
from jax.experimental.pallas import tpu_sc as plsc

_VOCAB = 100000
_DIM = 1024
_BATCH = 4
_SEQ = 4096
_NTOK = _BATCH * _SEQ

_NC = 2
_NS = 16
_NW = _NC * _NS
_TOK_PER_W = _NTOK // _NW
_CHUNK = 8
_NCHUNK = _TOK_PER_W // _CHUNK
_NBUF = 8
_NGROUP = _NCHUNK // _NBUF
_IDX_HEAD = 2 * _NBUF * _CHUNK


def _emb_body(idx_hbm, table_hbm, out_hbm, idx_v, rows_v, isem,
              gsem0, gsem1, gsem2, gsem3, gsem4, gsem5, gsem6, gsem7,
              ssem0, ssem1, ssem2, ssem3, ssem4, ssem5, ssem6, ssem7):
    wid = lax.axis_index("s") * _NC + lax.axis_index("c")
    wper = _SEQ // _TOK_PER_W
    row = wid // wper
    col = (wid % wper) * _TOK_PER_W
    pltpu.sync_copy(idx_hbm.at[row, pl.ds(col, _IDX_HEAD)],
                    idx_v.at[pl.ds(0, _IDX_HEAD)])
    idx_rest = pltpu.async_copy(
        idx_hbm.at[row, pl.ds(col + _IDX_HEAD, _TOK_PER_W - _IDX_HEAD)],
        idx_v.at[pl.ds(_IDX_HEAD, _TOK_PER_W - _IDX_HEAD)], isem)

    gsems = (gsem0, gsem1, gsem2, gsem3, gsem4, gsem5, gsem6, gsem7)
    ssems = (ssem0, ssem1, ssem2, ssem3, ssem4, ssem5, ssem6, ssem7)

    def gdesc(j, b):
        return pltpu.make_async_copy(
            table_hbm.at[idx_v.at[pl.ds(j * _CHUNK, _CHUNK)]],
            rows_v.at[b], gsems[b])

    def sdesc(j, b):
        return pltpu.make_async_copy(
            rows_v.at[b],
            out_hbm.at[row, pl.ds(col + j * _CHUNK, _CHUNK)],
            ssems[b])

    for b in range(_NBUF - 1):
        gdesc(b, b).start()
    idx_rest.wait()

    @pl.loop(0, _NGROUP)
    def _lap(g):
        j0 = g * _NBUF
        for b in range(_NBUF):
            j = j0 + b
            pb = (b - 1) % _NBUF
            gdesc(j, b).wait()
            if b == 0:
                @pl.when(g > 0)
                def _():
                    sdesc(j - 1, pb).wait()

                gdesc(j + _NBUF - 1, pb).start()
            else:
                sdesc(j - 1, pb).wait()

                @pl.when(g < _NGROUP - 1)
                def _():
                    gdesc(j + _NBUF - 1, pb).start()
            sdesc(j, b).start()

    sdesc(_NCHUNK - 1, (_NBUF - 1) % _NBUF).wait()


@functools.partial(jax.jit, static_argnames=())
def _emb(idx2d, weight):
    mesh = plsc.VectorSubcoreMesh(core_axis_name="c", subcore_axis_name="s")
    kern = pl.kernel(
        _emb_body,
        out_type=jax.ShapeDtypeStruct((_BATCH, _SEQ, _DIM), jnp.float32),
        mesh=mesh,
        scratch_types=[
            pltpu.VMEM((_TOK_PER_W,), jnp.int32),
            pltpu.VMEM((_NBUF, _CHUNK, _DIM), jnp.float32),
            pltpu.SemaphoreType.DMA,
            pltpu.SemaphoreType.DMA,
            pltpu.SemaphoreType.DMA,
            pltpu.SemaphoreType.DMA,
            pltpu.SemaphoreType.DMA,
            pltpu.SemaphoreType.DMA,
            pltpu.SemaphoreType.DMA,
            pltpu.SemaphoreType.DMA,
            pltpu.SemaphoreType.DMA,
            pltpu.SemaphoreType.DMA,
            pltpu.SemaphoreType.DMA,
            pltpu.SemaphoreType.DMA,
            pltpu.SemaphoreType.DMA,
            pltpu.SemaphoreType.DMA,
            pltpu.SemaphoreType.DMA,
            pltpu.SemaphoreType.DMA,
            pltpu.SemaphoreType.DMA,
        ],
    )
    return kern(idx2d, weight)


def kernel(indices, weight):
    return _emb(indices.astype(jnp.int32), weight)
